# baseline (speedup 1.0000x reference)
import jax
import jax.numpy as jnp
import numpy as np
from jax.experimental import pallas as pl
from jax.experimental.pallas import tpu as pltpu
from jax.sharding import Mesh, PartitionSpec as P


_TM = 256


def _mm_kernel(x_ref, w_ref, b_ref, o_ref):
    xb = x_ref[...].astype(jnp.bfloat16)
    acc = jnp.dot(xb, w_ref[...], preferred_element_type=jnp.float32)
    o_ref[...] = jnp.tanh(acc + b_ref[...])


def _mm_shard(x2, wb, b2):
    n, k = x2.shape
    m = wb.shape[1]
    tm = min(_TM, n)
    ni = pl.cdiv(n, tm)

    return pl.pallas_call(
        _mm_kernel,
        out_shape=jax.ShapeDtypeStruct((n, m), jnp.float32),
        grid=(ni,),
        in_specs=[
            pl.BlockSpec((tm, k), lambda i: (i, 0)),
            pl.BlockSpec((k, m), lambda i: (0, 0)),
            pl.BlockSpec((1, m), lambda i: (0, 0)),
        ],
        out_specs=pl.BlockSpec((tm, m), lambda i: (i, 0)),
        compiler_params=pltpu.CompilerParams(
            dimension_semantics=("parallel",),
            vmem_limit_bytes=64 * 1024 * 1024,
        ),
    )(x2, wb, b2)


def kernel(x, w_t, b2):
    in_ch = w_t.shape[0]
    x2 = x.reshape(-1, in_ch)
    wb = w_t.astype(jnp.bfloat16)

    devs = jax.devices()
    n_shards = 2 if (len(devs) >= 2 and x2.shape[0] % 2 == 0) else 1
    if n_shards == 1:
        return _mm_shard(x2, wb, b2)

    mesh = Mesh(np.array(devs[:n_shards]), ("dp",))
    f = jax.shard_map(
        _mm_shard,
        mesh=mesh,
        in_specs=(P("dp", None), P(None, None), P(None, None)),
        out_specs=P("dp", None),
        check_vma=False,
    )
    return f(x2, wb, b2)


# speedup vs baseline: 2.1675x; 2.1675x over previous
import functools

import jax
import jax.numpy as jnp
from jax.experimental import pallas as pl
from jax.experimental.pallas import tpu as pltpu


_TM = 256
_TC = 256


def _mm_kernel(w_ref, x_ref, b_ref, o_ref, wb_ref, *, nc, tc):
    i = pl.program_id(0)

    @pl.when(i < nc)
    def _():
        wb_ref[:, pl.ds(i * tc, tc)] = w_ref[...].astype(jnp.bfloat16)

    @pl.when(i >= nc)
    def _():
        xb = x_ref[...].astype(jnp.bfloat16)
        acc = jnp.dot(xb, wb_ref[...], preferred_element_type=jnp.float32)
        o_ref[...] = jnp.tanh(acc + b_ref[...])


@jax.jit
def _linear_tanh_fused(x2, w_t, b2):
    n, k = x2.shape
    m = w_t.shape[1]
    tm = min(_TM, n)
    tc = min(_TC, m)
    ni = pl.cdiv(n, tm)
    nc = pl.cdiv(m, tc)
    body = functools.partial(_mm_kernel, nc=nc, tc=tc)

    return pl.pallas_call(
        body,
        out_shape=jax.ShapeDtypeStruct((n, m), jnp.float32),
        grid=(nc + ni,),
        in_specs=[
            pl.BlockSpec((k, tc), lambda i: (0, jnp.minimum(i, nc - 1))),
            pl.BlockSpec((tm, k), lambda i: (jnp.maximum(i - nc, 0), 0)),
            pl.BlockSpec((1, m), lambda i: (0, 0)),
        ],
        out_specs=pl.BlockSpec(
            (tm, m), lambda i: (jnp.maximum(i - nc, 0), 0)
        ),
        scratch_shapes=[pltpu.VMEM((k, m), jnp.bfloat16)],
        compiler_params=pltpu.CompilerParams(
            dimension_semantics=("arbitrary",),
            vmem_limit_bytes=64 * 1024 * 1024,
        ),
    )(w_t, x2, b2)


def kernel(x, w_t, b2):
    in_ch = w_t.shape[0]
    x2 = x.reshape(-1, in_ch)
    return _linear_tanh_fused(x2, w_t, b2)
